# Initial kernel scaffold; baseline (speedup 1.0000x reference)
import jax
import jax.numpy as jnp
from jax.experimental import pallas as pl


def kernel(msg, index, t, dim_size):
    raise NotImplementedError("write your pallas kernel here")


# speedup vs baseline: 30.3418x; 30.3418x over previous
import jax
import jax.numpy as jnp
from jax import lax
from jax.experimental import pallas as pl
from jax.experimental.pallas import tpu as pltpu
from jax.experimental.pallas import tpu_sc as plsc

_E = 10000
_D = 256
_N = 10000
_L = 16
_NC, _NS = 2, 16
_NW = _NC * _NS
_NPT = 320
_TAIL = _N - (_NW - 1) * _NPT
_GCH = 64
_NEG_INF = float("-inf")


def _sc_body(msg_hbm, idx_hbm, t_hbm, out_hbm,
             idx_v, t_v, best_t, best_e, scr, gidx0, gidx1, gidx2, gidx3, gidx4, rows_v, sem):
    wid = lax.axis_index("s") * _NC + lax.axis_index("c")
    lo = wid * _NPT
    lane = lax.iota(jnp.int32, _L)

    with jax.named_scope("ph_stage"):
        pltpu.sync_copy(idx_hbm, idx_v)
        pltpu.sync_copy(t_hbm, t_v)

    neg = jnp.full((_L,), _NEG_INF, jnp.float32)
    zero_i = jnp.zeros((_L,), jnp.int32)

    def init_body(j, carry):
        sl = pl.ds(j * _L, _L)
        best_t[sl] = neg
        best_e[sl] = zero_i
        return carry

    lax.fori_loop(0, _NPT // _L, init_body, 0)

    def scan_body(i, carry):
        sl = pl.ds(i * _L, _L)
        idxv = idx_v[sl]
        tv = t_v[sl]
        ev = lane + i * _L
        local_raw = idxv - lo
        m0 = (local_raw >= 0) & (local_raw < _NPT)
        local = jnp.where(m0, local_raw, 0)

        def cond(mi):
            return jnp.max(mi) > 0

        def round_body(mi):
            mb = mi > 0
            plsc.store_scatter(scr, [local], lane, mask=mb)
            rb = plsc.load_gather(scr, [local], mask=mb)
            winner = mb & (rb == lane)
            curt = plsc.load_gather(best_t, [local], mask=winner)
            cure = plsc.load_gather(best_e, [local], mask=winner)
            win = winner & ((tv > curt) | ((tv == curt) & (ev < cure)))
            plsc.store_scatter(best_t, [local], tv, mask=win)
            plsc.store_scatter(best_e, [local], ev, mask=win)
            return jnp.where(mb & ~winner, mi, 0)

        lax.while_loop(cond, round_body, m0.astype(jnp.int32))
        return carry

    with jax.named_scope("ph_scan"):
        lax.fori_loop(0, _E // _L, scan_body, 0)

    gidxs = [gidx0, gidx1, gidx2, gidx3, gidx4]
    for j in range(_NPT // _L):
        sl = pl.ds(j * _L, _L)
        valid = best_t[sl] > _NEG_INF
        spread = jnp.minimum(lane + (lo + j * _L), _E - 1)
        gidxs[j // 4][pl.ds((j % 4) * _L, _L)] = jnp.where(
            valid, best_e[sl], spread)

    with jax.named_scope("ph_gather"):
        copies = []
        for ch in range(_NPT // _GCH):
            sl = pl.ds(ch * _GCH, _GCH)
            copies.append(
                pltpu.async_copy(msg_hbm.at[gidxs[ch]], rows_v.at[sl], sem))
        for c in copies:
            c.wait()

    zrow = jnp.zeros((_L,), jnp.float32)

    def zero_body(g, carry):
        inval = (best_t[pl.ds(g * _L, _L)] == _NEG_INF).astype(jnp.int32)
        for k in range(_L):
            @pl.when(inval[k] > 0)
            def _():
                for q in range(_D // _L):
                    rows_v[g * _L + k, pl.ds(q * _L, _L)] = zrow
        return carry

    with jax.named_scope("ph_zero"):
        lax.fori_loop(0, _NPT // _L, zero_body, 0)

    @pl.when(wid < _NW - 1)
    def _():
        pltpu.sync_copy(rows_v, out_hbm.at[pl.ds(lo, _NPT)])

    @pl.when(wid == _NW - 1)
    def _():
        pltpu.sync_copy(rows_v.at[pl.ds(0, _TAIL)], out_hbm.at[pl.ds(lo, _TAIL)])


def _build():
    mesh = plsc.VectorSubcoreMesh(core_axis_name="c", subcore_axis_name="s")
    return pl.kernel(
        _sc_body,
        out_type=jax.ShapeDtypeStruct((_N, _D), jnp.float32),
        mesh=mesh,
        compiler_params=pltpu.CompilerParams(needs_layout_passes=False),
        scratch_types=[
            pltpu.VMEM((_E,), jnp.int32),
            pltpu.VMEM((_E,), jnp.float32),
            pltpu.VMEM((_NPT,), jnp.float32),
            pltpu.VMEM((_NPT,), jnp.int32),
            pltpu.VMEM((_NPT,), jnp.int32),
            pltpu.VMEM((_GCH,), jnp.int32),
            pltpu.VMEM((_GCH,), jnp.int32),
            pltpu.VMEM((_GCH,), jnp.int32),
            pltpu.VMEM((_GCH,), jnp.int32),
            pltpu.VMEM((_GCH,), jnp.int32),
            pltpu.VMEM((_NPT, _D), jnp.float32),
            pltpu.SemaphoreType.DMA,
        ],
    )


def kernel(msg, index, t, dim_size):
    del dim_size
    return _build()(msg, index.astype(jnp.int32), t.astype(jnp.float32))


# speedup vs baseline: 31.6293x; 1.0424x over previous
import jax
import jax.numpy as jnp
from jax import lax
from jax.experimental import pallas as pl
from jax.experimental.pallas import tpu as pltpu
from jax.experimental.pallas import tpu_sc as plsc

_E = 10000
_D = 256
_N = 10000
_L = 16
_NC, _NS = 2, 16
_NW = _NC * _NS
_NPT = 320
_TAIL = _N - (_NW - 1) * _NPT
_GCH = 64
_NEG_INF = float("-inf")


def _sc_body(msg_hbm, idx_hbm, t_hbm, out_hbm,
             idx_v, t_v, best_t, best_e, scr, gidx0, gidx1, gidx2, gidx3, gidx4, rows_v, sem):
    wid = lax.axis_index("s") * _NC + lax.axis_index("c")
    lo = wid * _NPT
    lane = lax.iota(jnp.int32, _L)

    with jax.named_scope("ph_stage"):
        pltpu.sync_copy(idx_hbm, idx_v)
        pltpu.sync_copy(t_hbm, t_v)

    neg = jnp.full((_L,), _NEG_INF, jnp.float32)
    zero_i = jnp.zeros((_L,), jnp.int32)

    def init_body(j, carry):
        sl = pl.ds(j * _L, _L)
        best_t[sl] = neg
        best_e[sl] = zero_i
        return carry

    lax.fori_loop(0, _NPT // _L, init_body, 0)

    def scan_body(i, carry):
        sl = pl.ds(i * _L, _L)
        idxv = idx_v[sl]
        tv = t_v[sl]
        ev = lane + i * _L
        local_raw = idxv - lo
        m0 = (local_raw >= 0) & (local_raw < _NPT)
        local = jnp.where(m0, local_raw, 0)

        def cond(mi):
            return plsc.all_reduce_population_count(mi > 0)[0] > 0

        def round_body(mi):
            mb = mi > 0
            plsc.store_scatter(scr, [local], lane, mask=mb)
            rb = plsc.load_gather(scr, [local], mask=mb)
            winner = mb & (rb == lane)
            curt = plsc.load_gather(best_t, [local], mask=winner)
            cure = plsc.load_gather(best_e, [local], mask=winner)
            win = winner & ((tv > curt) | ((tv == curt) & (ev < cure)))
            plsc.store_scatter(best_t, [local], tv, mask=win)
            plsc.store_scatter(best_e, [local], ev, mask=win)
            return jnp.where(mb & ~winner, mi, 0)

        lax.while_loop(cond, round_body, m0.astype(jnp.int32))
        return carry

    with jax.named_scope("ph_scan"):
        lax.fori_loop(0, _E // _L, scan_body, 0)

    gidxs = [gidx0, gidx1, gidx2, gidx3, gidx4]
    for j in range(_NPT // _L):
        sl = pl.ds(j * _L, _L)
        valid = best_t[sl] > _NEG_INF
        spread = jnp.minimum(lane + (lo + j * _L), _E - 1)
        gidxs[j // 4][pl.ds((j % 4) * _L, _L)] = jnp.where(
            valid, best_e[sl], spread)

    with jax.named_scope("ph_gather"):
        copies = []
        for ch in range(_NPT // _GCH):
            sl = pl.ds(ch * _GCH, _GCH)
            copies.append(
                pltpu.async_copy(msg_hbm.at[gidxs[ch]], rows_v.at[sl], sem))
        for c in copies:
            c.wait()

    zrow = jnp.zeros((_L,), jnp.float32)

    def zero_body(g, carry):
        inval = (best_t[pl.ds(g * _L, _L)] == _NEG_INF).astype(jnp.int32)
        for k in range(_L):
            @pl.when(inval[k] > 0)
            def _():
                for q in range(_D // _L):
                    rows_v[g * _L + k, pl.ds(q * _L, _L)] = zrow
        return carry

    with jax.named_scope("ph_zero"):
        lax.fori_loop(0, _NPT // _L, zero_body, 0)

    @pl.when(wid < _NW - 1)
    def _():
        pltpu.sync_copy(rows_v, out_hbm.at[pl.ds(lo, _NPT)])

    @pl.when(wid == _NW - 1)
    def _():
        pltpu.sync_copy(rows_v.at[pl.ds(0, _TAIL)], out_hbm.at[pl.ds(lo, _TAIL)])


def _build():
    mesh = plsc.VectorSubcoreMesh(core_axis_name="c", subcore_axis_name="s")
    return pl.kernel(
        _sc_body,
        out_type=jax.ShapeDtypeStruct((_N, _D), jnp.float32),
        mesh=mesh,
        compiler_params=pltpu.CompilerParams(needs_layout_passes=False),
        scratch_types=[
            pltpu.VMEM((_E,), jnp.int32),
            pltpu.VMEM((_E,), jnp.float32),
            pltpu.VMEM((_NPT,), jnp.float32),
            pltpu.VMEM((_NPT,), jnp.int32),
            pltpu.VMEM((_NPT,), jnp.int32),
            pltpu.VMEM((_GCH,), jnp.int32),
            pltpu.VMEM((_GCH,), jnp.int32),
            pltpu.VMEM((_GCH,), jnp.int32),
            pltpu.VMEM((_GCH,), jnp.int32),
            pltpu.VMEM((_GCH,), jnp.int32),
            pltpu.VMEM((_NPT, _D), jnp.float32),
            pltpu.SemaphoreType.DMA,
        ],
    )


def kernel(msg, index, t, dim_size):
    del dim_size
    return _build()(msg, index.astype(jnp.int32), t.astype(jnp.float32))


# speedup vs baseline: 32.1068x; 1.0151x over previous
import jax
import jax.numpy as jnp
from jax import lax
from jax.experimental import pallas as pl
from jax.experimental.pallas import tpu as pltpu
from jax.experimental.pallas import tpu_sc as plsc

_E = 10000
_D = 256
_N = 10000
_L = 16
_NC, _NS = 2, 16
_NW = _NC * _NS
_NPT = 320
_TAIL = _N - (_NW - 1) * _NPT
_GCH = 64
_NEG_INF = float("-inf")


def _sc_body(msg_hbm, idx_hbm, t_hbm, out_hbm,
             idx_v, t_v, best_t, best_e, scr, gidx0, gidx1, gidx2, gidx3, gidx4, rows_v, sem):
    wid = lax.axis_index("s") * _NC + lax.axis_index("c")
    lo = wid * _NPT
    lane = lax.iota(jnp.int32, _L)

    with jax.named_scope("ph_stage"):
        pltpu.sync_copy(idx_hbm, idx_v)
        pltpu.sync_copy(t_hbm, t_v)

    neg = jnp.full((_L,), _NEG_INF, jnp.float32)
    zero_i = jnp.zeros((_L,), jnp.int32)

    def init_body(j, carry):
        sl = pl.ds(j * _L, _L)
        best_t[sl] = neg
        best_e[sl] = zero_i
        return carry

    lax.fori_loop(0, _NPT // _L, init_body, 0)

    def scan_once(i):
        sl = pl.ds(i * _L, _L)
        idxv = idx_v[sl]
        tv = t_v[sl]
        ev = lane + i * _L
        local_raw = idxv - lo
        m0 = (local_raw >= 0) & (local_raw < _NPT)
        local = jnp.where(m0, local_raw, 0)

        def cond(mi):
            return plsc.all_reduce_population_count(mi > 0)[0] > 0

        def round_body(mi):
            mb = mi > 0
            plsc.store_scatter(scr, [local], lane, mask=mb)
            rb = plsc.load_gather(scr, [local], mask=mb)
            winner = mb & (rb == lane)
            curt = plsc.load_gather(best_t, [local], mask=winner)
            cure = plsc.load_gather(best_e, [local], mask=winner)
            win = winner & ((tv > curt) | ((tv == curt) & (ev < cure)))
            plsc.store_scatter(best_t, [local], tv, mask=win)
            plsc.store_scatter(best_e, [local], ev, mask=win)
            return jnp.where(mb & ~winner, mi, 0)

        lax.while_loop(cond, round_body, m0.astype(jnp.int32))

    def scan_body(i, carry):
        scan_once(i * 2)
        scan_once(i * 2 + 1)
        return carry

    with jax.named_scope("ph_scan"):
        lax.fori_loop(0, _E // _L // 2, scan_body, 0)

    gidxs = [gidx0, gidx1, gidx2, gidx3, gidx4]
    for j in range(_NPT // _L):
        sl = pl.ds(j * _L, _L)
        valid = best_t[sl] > _NEG_INF
        spread = jnp.minimum(lane + (lo + j * _L), _E - 1)
        gidxs[j // 4][pl.ds((j % 4) * _L, _L)] = jnp.where(
            valid, best_e[sl], spread)

    with jax.named_scope("ph_gather"):
        copies = []
        for ch in range(_NPT // _GCH):
            sl = pl.ds(ch * _GCH, _GCH)
            copies.append(
                pltpu.async_copy(msg_hbm.at[gidxs[ch]], rows_v.at[sl], sem))
        for c in copies:
            c.wait()

    zrow = jnp.zeros((_L,), jnp.float32)

    def zero_body(g, carry):
        inval = (best_t[pl.ds(g * _L, _L)] == _NEG_INF).astype(jnp.int32)
        for k in range(_L):
            @pl.when(inval[k] > 0)
            def _():
                for q in range(_D // _L):
                    rows_v[g * _L + k, pl.ds(q * _L, _L)] = zrow
        return carry

    with jax.named_scope("ph_zero"):
        lax.fori_loop(0, _NPT // _L, zero_body, 0)

    @pl.when(wid < _NW - 1)
    def _():
        pltpu.sync_copy(rows_v, out_hbm.at[pl.ds(lo, _NPT)])

    @pl.when(wid == _NW - 1)
    def _():
        pltpu.sync_copy(rows_v.at[pl.ds(0, _TAIL)], out_hbm.at[pl.ds(lo, _TAIL)])


def _build():
    mesh = plsc.VectorSubcoreMesh(core_axis_name="c", subcore_axis_name="s")
    return pl.kernel(
        _sc_body,
        out_type=jax.ShapeDtypeStruct((_N, _D), jnp.float32),
        mesh=mesh,
        compiler_params=pltpu.CompilerParams(needs_layout_passes=False),
        scratch_types=[
            pltpu.VMEM((_E,), jnp.int32),
            pltpu.VMEM((_E,), jnp.float32),
            pltpu.VMEM((_NPT,), jnp.float32),
            pltpu.VMEM((_NPT,), jnp.int32),
            pltpu.VMEM((_NPT,), jnp.int32),
            pltpu.VMEM((_GCH,), jnp.int32),
            pltpu.VMEM((_GCH,), jnp.int32),
            pltpu.VMEM((_GCH,), jnp.int32),
            pltpu.VMEM((_GCH,), jnp.int32),
            pltpu.VMEM((_GCH,), jnp.int32),
            pltpu.VMEM((_NPT, _D), jnp.float32),
            pltpu.SemaphoreType.DMA,
        ],
    )


def kernel(msg, index, t, dim_size):
    del dim_size
    return _build()(msg, index.astype(jnp.int32), t.astype(jnp.float32))
